# Initial kernel scaffold; baseline (speedup 1.0000x reference)
import jax
import jax.numpy as jnp
from jax.experimental import pallas as pl


def kernel(node_idx, ent_table):
    raise NotImplementedError("write your pallas kernel here")


# speedup vs baseline: 1.8303x; 1.8303x over previous
import functools

import jax
import jax.numpy as jnp
from jax import lax
from jax.experimental import pallas as pl
from jax.experimental.pallas import tpu as pltpu
from jax.experimental.pallas import tpu_sc as plsc

_NUM_CORES = 2
_NUM_SUBCORES = 16
_NW = _NUM_CORES * _NUM_SUBCORES

_ENT_DIM = 64
_IDX_MINOR = 128
_SUB = 4
_CHUNK = _IDX_MINOR * _SUB


def _gather_body(n_chunks, idx_hbm, table_hbm, out_hbm, idx_v, rows, gsem, osem):
    wid = lax.axis_index("s") * _NUM_CORES + lax.axis_index("c")
    row_base = wid * (n_chunks * _CHUNK)

    pltpu.sync_copy(idx_hbm.at[wid], idx_v)

    del osem

    @pl.loop(0, n_chunks)
    def _(g):
        for t in range(_SUB):
            pltpu.async_copy(
                table_hbm.at[idx_v.at[g * _SUB + t]],
                rows.at[0, pl.ds(t * _IDX_MINOR, _IDX_MINOR)],
                gsem.at[0],
            )
        for t in range(_SUB):
            pltpu.make_async_copy(
                table_hbm.at[idx_v.at[g * _SUB + t]],
                rows.at[0, pl.ds(t * _IDX_MINOR, _IDX_MINOR)],
                gsem.at[0],
            ).wait()
        pltpu.sync_copy(
            rows.at[0], out_hbm.at[pl.ds(row_base + g * _CHUNK, _CHUNK)]
        )


def kernel(node_idx, ent_table):
    b, l = node_idx.shape
    n_rows = b * l
    assert n_rows % (_NW * _CHUNK) == 0
    n_chunks = n_rows // (_NW * _CHUNK)

    idx = node_idx.reshape(_NW, n_chunks * _SUB, _IDX_MINOR).astype(jnp.int32)

    mesh = plsc.VectorSubcoreMesh(
        core_axis_name="c", subcore_axis_name="s",
        num_cores=_NUM_CORES, num_subcores=_NUM_SUBCORES,
    )
    flat_out = pl.kernel(
        functools.partial(_gather_body, n_chunks),
        out_type=jax.ShapeDtypeStruct((n_rows, _ENT_DIM), jnp.float32),
        mesh=mesh,
        scratch_types=[
            pltpu.VMEM((n_chunks * _SUB, _IDX_MINOR), jnp.int32),
            pltpu.VMEM((2, _CHUNK, _ENT_DIM), jnp.float32),
            pltpu.SemaphoreType.DMA((2,)),
            pltpu.SemaphoreType.DMA((2,)),
        ],
        compiler_params=pltpu.CompilerParams(use_tc_tiling_on_sc=False),
        name="sc_embedding_gather",
    )(idx, ent_table)

    return flat_out.reshape(b, l, _ENT_DIM)


# speedup vs baseline: 1.8742x; 1.0240x over previous
import functools

import jax
import jax.numpy as jnp
from jax import lax
from jax.experimental import pallas as pl
from jax.experimental.pallas import tpu as pltpu
from jax.experimental.pallas import tpu_sc as plsc

_NUM_CORES = 2
_NUM_SUBCORES = 16
_NW = _NUM_CORES * _NUM_SUBCORES

_ENT_DIM = 64
_IDX_MINOR = 128
_SUB = 5
_CHUNK = _IDX_MINOR * _SUB


def _gather_body(n_chunks, idx_hbm, table_hbm, out_hbm, idx_v, rows, gsem, osem):
    wid = lax.axis_index("s") * _NUM_CORES + lax.axis_index("c")
    row_base = wid * (n_chunks * _CHUNK)

    pltpu.sync_copy(idx_hbm.at[wid], idx_v)

    def fire_gathers(g, b):
        for t in range(_SUB):
            pltpu.async_copy(
                table_hbm.at[idx_v.at[g * _SUB + t]],
                rows.at[b, pl.ds(t * _IDX_MINOR, _IDX_MINOR)],
                gsem.at[b],
            )

    def drain_gathers(g, b):
        for t in range(_SUB):
            pltpu.make_async_copy(
                table_hbm.at[idx_v.at[g * _SUB + t]],
                rows.at[b, pl.ds(t * _IDX_MINOR, _IDX_MINOR)],
                gsem.at[b],
            ).wait()

    def fire_out(g, b):
        pltpu.async_copy(
            rows.at[b],
            out_hbm.at[pl.ds(row_base + g * _CHUNK, _CHUNK)],
            osem.at[b],
        )

    def drain_out(g, b):
        pltpu.make_async_copy(
            rows.at[b],
            out_hbm.at[pl.ds(row_base + g * _CHUNK, _CHUNK)],
            osem.at[b],
        ).wait()

    fire_gathers(0, 0)

    @pl.loop(0, n_chunks, step=2)
    def _(g):
        for b in range(2):
            gb = g + b
            nxt = 1 - b

            @pl.when(gb >= 1)
            def _():
                drain_out(gb - 1, nxt)

            @pl.when(gb + 1 < n_chunks)
            def _():
                fire_gathers(gb + 1, nxt)

            drain_gathers(gb, b)
            fire_out(gb, b)

    drain_out(n_chunks - 1, 1)


def kernel(node_idx, ent_table):
    b, l = node_idx.shape
    n_rows = b * l
    assert n_rows % (_NW * _CHUNK) == 0
    n_chunks = n_rows // (_NW * _CHUNK)
    assert n_chunks % 2 == 0

    idx = node_idx.reshape(_NW, n_chunks * _SUB, _IDX_MINOR).astype(jnp.int32)

    mesh = plsc.VectorSubcoreMesh(
        core_axis_name="c", subcore_axis_name="s",
        num_cores=_NUM_CORES, num_subcores=_NUM_SUBCORES,
    )
    flat_out = pl.kernel(
        functools.partial(_gather_body, n_chunks),
        out_type=jax.ShapeDtypeStruct((n_rows, _ENT_DIM), jnp.float32),
        mesh=mesh,
        scratch_types=[
            pltpu.VMEM((n_chunks * _SUB, _IDX_MINOR), jnp.int32),
            pltpu.VMEM((2, _CHUNK, _ENT_DIM), jnp.float32),
            pltpu.SemaphoreType.DMA((2,)),
            pltpu.SemaphoreType.DMA((2,)),
        ],
        compiler_params=pltpu.CompilerParams(use_tc_tiling_on_sc=False),
        name="sc_embedding_gather",
    )(idx, ent_table)

    return flat_out.reshape(b, l, _ENT_DIM)
